# baseline (speedup 1.0000x reference)
import jax
import jax.numpy as jnp
from jax import lax
from jax.experimental import pallas as pl
from jax.experimental.pallas import tpu as pltpu
from jax.experimental.pallas import tpu_sc as plsc

N_FRAMES = 1024
N_Q, BINS, DIM = 8, 512, 32
F_TILE = 1024

NUM_CORES = 2
NUM_SUBCORES = 16
NUM_WORKERS = NUM_CORES * NUM_SUBCORES
N_GATHER = 2 * N_FRAMES * N_Q
K_PER_W = N_GATHER // NUM_WORKERS
TABLE_W = 128


def _dsum(t):
    return jnp.sum(t, axis=-1, keepdims=True)


def _top2_body(x_ref, cbt_ref, i1_ref, i2_ref, if1_ref, if2_ref):
    xv = x_ref[...]
    cbt = cbt_ref[...]
    cn2 = jnp.sum(cbt * cbt, axis=0, keepdims=True)
    xc = lax.dot_general(xv, cbt, (((1,), (0,)), ((), ())),
                         precision=lax.Precision.HIGHEST,
                         preferred_element_type=jnp.float32)
    scores = cn2 - 2.0 * xc
    iota = lax.broadcasted_iota(jnp.int32, (F_TILE, BINS), 1)
    i1c, i2c, if1c, if2c = [], [], [], []
    for q in range(N_Q):
        sq = scores[:, q * BINS:(q + 1) * BINS]
        m1 = jnp.min(sq, axis=-1, keepdims=True)
        i1 = jnp.min(jnp.where(sq == m1, iota, BINS), axis=-1, keepdims=True)
        sq2 = jnp.where(iota == i1, jnp.inf, sq)
        m2 = jnp.min(sq2, axis=-1, keepdims=True)
        i2 = jnp.min(jnp.where(sq2 == m2, iota, BINS), axis=-1, keepdims=True)
        i1c.append(i1)
        i2c.append(i2)
        if1c.append(i1 + q * BINS)
        if2c.append(i2 + q * BINS)
    i1_ref[...] = jnp.concatenate(i1c, axis=1)
    i2_ref[...] = jnp.concatenate(i2c, axis=1)
    if1_ref[...] = jnp.concatenate(if1c, axis=1)
    if2_ref[...] = jnp.concatenate(if2c, axis=1)


def _top2_call(x_flat, cb_t):
    out = jax.ShapeDtypeStruct((N_FRAMES, N_Q), jnp.int32)
    return pl.pallas_call(
        _top2_body,
        grid=(N_FRAMES // F_TILE,),
        in_specs=[
            pl.BlockSpec((F_TILE, DIM), lambda i: (i, 0)),
            pl.BlockSpec((DIM, N_Q * BINS), lambda i: (0, 0)),
        ],
        out_specs=[pl.BlockSpec((F_TILE, N_Q), lambda i: (i, 0))] * 4,
        out_shape=[out] * 4,
        compiler_params=pltpu.CompilerParams(
            dimension_semantics=("arbitrary",)),
    )(x_flat, cb_t)


def _gather_body(table_hbm, idx_hbm, out_hbm, idx_v, rows_v, sem):
    wid = lax.axis_index("s") * NUM_CORES + lax.axis_index("c")
    base = wid * K_PER_W
    pltpu.sync_copy(idx_hbm.at[pl.ds(base, K_PER_W)], idx_v)
    pltpu.async_copy(table_hbm.at[idx_v], rows_v, sem).wait()
    pltpu.sync_copy(rows_v, out_hbm.at[pl.ds(base, K_PER_W)])


def _gather_call(table, idx_flat):
    mesh = plsc.VectorSubcoreMesh(core_axis_name="c", subcore_axis_name="s")
    return pl.kernel(
        _gather_body,
        mesh=mesh,
        out_type=jax.ShapeDtypeStruct((N_GATHER, DIM), jnp.float32),
        scratch_types=[
            pltpu.VMEM((K_PER_W,), jnp.int32),
            pltpu.VMEM((K_PER_W, DIM), jnp.float32),
            pltpu.SemaphoreType.DMA,
        ],
        compiler_params=pltpu.CompilerParams(use_tc_tiling_on_sc=False),
    )(table, idx_flat)


def _rescue_body(x_ref, r1_ref, r2_ref, i1_ref, i2_ref,
                 idx_ref, qst_ref, loss_ref):
    xv = x_ref[...]
    idx_cols = []
    quant = None
    for q in range(N_Q):
        c1 = r1_ref[q]
        c2 = r2_ref[q]
        t1 = xv - c1
        t2 = xv - c2
        d1 = _dsum(t1 * t1)
        d2 = _dsum(t2 * t2)
        i1 = i1_ref[:, q:q + 1]
        i2 = i2_ref[:, q:q + 1]
        use1 = (d1 < d2) | ((d1 == d2) & (i1 < i2))
        idx_cols.append(jnp.where(use1, i1, i2))
        csel = jnp.where(use1, c1, c2)
        quant = csel if quant is None else quant + csel
    idx_ref[...] = jnp.concatenate(idx_cols, axis=1)
    dlt = quant - xv
    qst_ref[...] = xv + dlt
    loss_ref[0, 0] = jnp.sum(dlt * dlt) * (2.0 / (N_FRAMES * DIM))


def _rescue_call(x_flat, rows1, rows2, i1, i2):
    full = lambda s: pl.BlockSpec(s, lambda: tuple(0 for _ in s))
    return pl.pallas_call(
        _rescue_body,
        in_specs=[
            full((N_FRAMES, DIM)),
            full((N_Q, N_FRAMES, DIM)),
            full((N_Q, N_FRAMES, DIM)),
            full((N_FRAMES, N_Q)),
            full((N_FRAMES, N_Q)),
        ],
        out_specs=[
            full((N_FRAMES, N_Q)),
            full((N_FRAMES, DIM)),
            pl.BlockSpec(memory_space=pltpu.SMEM),
        ],
        out_shape=[
            jax.ShapeDtypeStruct((N_FRAMES, N_Q), jnp.int32),
            jax.ShapeDtypeStruct((N_FRAMES, DIM), jnp.float32),
            jax.ShapeDtypeStruct((1, 1), jnp.float32),
        ],
    )(x_flat, rows1, rows2, i1, i2)


def kernel(x, sample_rate, bandwidth, codebook):
    b, c, t = x.shape
    x_flat = jnp.transpose(x, (0, 2, 1)).reshape(-1, c)
    cb_t = jnp.transpose(codebook, (2, 0, 1)).reshape(DIM, N_Q * BINS)
    i1, i2, if1, if2 = _top2_call(x_flat, cb_t)
    idx_flat = jnp.concatenate(
        [jnp.transpose(if1).reshape(-1), jnp.transpose(if2).reshape(-1)])
    table = codebook.reshape(N_Q * BINS, DIM)
    rows = _gather_call(table, idx_flat)
    rows1 = rows[:N_FRAMES * N_Q].reshape(N_Q, N_FRAMES, DIM)
    rows2 = rows[N_FRAMES * N_Q:].reshape(N_Q, N_FRAMES, DIM)
    indices, qst_flat, loss11 = _rescue_call(x_flat, rows1, rows2, i1, i2)
    quantized_st = jnp.transpose(qst_flat.reshape(b, t, c), (0, 2, 1))
    return quantized_st, indices, loss11[0, 0]


# speedup vs baseline: 1.0097x; 1.0097x over previous
import jax
import jax.numpy as jnp
from jax import lax
from jax.experimental import pallas as pl
from jax.experimental.pallas import tpu as pltpu
from jax.experimental.pallas import tpu_sc as plsc

N_FRAMES = 1024
N_Q, BINS, DIM = 8, 512, 32
F_TILE = 512

NUM_CORES = 2
NUM_SUBCORES = 16
NUM_WORKERS = NUM_CORES * NUM_SUBCORES
N_GATHER = 2 * N_FRAMES * N_Q
K_PER_W = N_GATHER // NUM_WORKERS


def _dsum(t):
    return jnp.sum(t, axis=-1, keepdims=True)


def _top2_body(x_ref, cbt_ref, i1_ref, i2_ref, if1_ref, if2_ref):
    xv = x_ref[...]
    cbt = cbt_ref[...]
    cn2 = jnp.sum(cbt * cbt, axis=0, keepdims=True)
    xc = lax.dot_general(xv, cbt, (((1,), (0,)), ((), ())),
                         precision=lax.Precision.HIGHEST,
                         preferred_element_type=jnp.float32)
    scores = cn2 - 2.0 * xc
    iota = lax.broadcasted_iota(jnp.int32, (F_TILE, BINS), 1)
    i1c, i2c, if1c, if2c = [], [], [], []
    for q in range(N_Q):
        sq = scores[:, q * BINS:(q + 1) * BINS]
        m1 = jnp.min(sq, axis=-1, keepdims=True)
        i1 = jnp.min(jnp.where(sq == m1, iota, BINS), axis=-1, keepdims=True)
        sq2 = jnp.where(iota == i1, jnp.inf, sq)
        m2 = jnp.min(sq2, axis=-1, keepdims=True)
        i2 = jnp.min(jnp.where(sq2 == m2, iota, BINS), axis=-1, keepdims=True)
        i1c.append(i1)
        i2c.append(i2)
        if1c.append(i1 + q * BINS)
        if2c.append(i2 + q * BINS)
    i1_ref[...] = jnp.concatenate(i1c, axis=1)
    i2_ref[...] = jnp.concatenate(i2c, axis=1)
    if1_ref[...] = jnp.concatenate(if1c, axis=1)
    if2_ref[...] = jnp.concatenate(if2c, axis=1)


def _top2_call(x_flat, cb_t):
    out = jax.ShapeDtypeStruct((N_FRAMES, N_Q), jnp.int32)
    return pl.pallas_call(
        _top2_body,
        grid=(N_FRAMES // F_TILE,),
        in_specs=[
            pl.BlockSpec((F_TILE, DIM), lambda i: (i, 0)),
            pl.BlockSpec((DIM, N_Q * BINS), lambda i: (0, 0)),
        ],
        out_specs=[pl.BlockSpec((F_TILE, N_Q), lambda i: (i, 0))] * 4,
        out_shape=[out] * 4,
        compiler_params=pltpu.CompilerParams(
            dimension_semantics=("arbitrary",)),
    )(x_flat, cb_t)


def _gather_body(table_hbm, idx_hbm, out_hbm, idx_v, rows_v, sem):
    wid = lax.axis_index("s") * NUM_CORES + lax.axis_index("c")
    base = wid * K_PER_W
    pltpu.sync_copy(idx_hbm.at[pl.ds(base, K_PER_W)], idx_v)
    pltpu.async_copy(table_hbm.at[idx_v], rows_v, sem).wait()
    pltpu.sync_copy(rows_v, out_hbm.at[pl.ds(base, K_PER_W)])


def _gather_call(table, idx_flat):
    mesh = plsc.VectorSubcoreMesh(core_axis_name="c", subcore_axis_name="s")
    return pl.kernel(
        _gather_body,
        mesh=mesh,
        out_type=jax.ShapeDtypeStruct((N_GATHER, DIM), jnp.float32),
        scratch_types=[
            pltpu.VMEM((K_PER_W,), jnp.int32),
            pltpu.VMEM((K_PER_W, DIM), jnp.float32),
            pltpu.SemaphoreType.DMA,
        ],
        compiler_params=pltpu.CompilerParams(use_tc_tiling_on_sc=False),
    )(table, idx_flat)


def _rescue_body(x_ref, r1_ref, r2_ref, i1_ref, i2_ref,
                 idx_ref, qst_ref, loss_ref):
    xv = x_ref[...]
    idx_cols = []
    quant = None
    for q in range(N_Q):
        c1 = r1_ref[q]
        c2 = r2_ref[q]
        t1 = xv - c1
        t2 = xv - c2
        d1 = _dsum(t1 * t1)
        d2 = _dsum(t2 * t2)
        i1 = i1_ref[:, q:q + 1]
        i2 = i2_ref[:, q:q + 1]
        use1 = (d1 < d2) | ((d1 == d2) & (i1 < i2))
        idx_cols.append(jnp.where(use1, i1, i2))
        csel = jnp.where(use1, c1, c2)
        quant = csel if quant is None else quant + csel
    idx_ref[...] = jnp.concatenate(idx_cols, axis=1)
    dlt = quant - xv
    qst_ref[...] = xv + dlt
    loss_ref[0, 0] = jnp.sum(dlt * dlt) * (2.0 / (N_FRAMES * DIM))


def _rescue_call(x_flat, rows1, rows2, i1, i2):
    full = lambda s: pl.BlockSpec(s, lambda: tuple(0 for _ in s))
    return pl.pallas_call(
        _rescue_body,
        in_specs=[
            full((N_FRAMES, DIM)),
            full((N_Q, N_FRAMES, DIM)),
            full((N_Q, N_FRAMES, DIM)),
            full((N_FRAMES, N_Q)),
            full((N_FRAMES, N_Q)),
        ],
        out_specs=[
            full((N_FRAMES, N_Q)),
            full((N_FRAMES, DIM)),
            pl.BlockSpec(memory_space=pltpu.SMEM),
        ],
        out_shape=[
            jax.ShapeDtypeStruct((N_FRAMES, N_Q), jnp.int32),
            jax.ShapeDtypeStruct((N_FRAMES, DIM), jnp.float32),
            jax.ShapeDtypeStruct((1, 1), jnp.float32),
        ],
    )(x_flat, rows1, rows2, i1, i2)


def kernel(x, sample_rate, bandwidth, codebook):
    b, c, t = x.shape
    x_flat = jnp.transpose(x, (0, 2, 1)).reshape(-1, c)
    cb_t = jnp.transpose(codebook, (2, 0, 1)).reshape(DIM, N_Q * BINS)
    i1, i2, if1, if2 = _top2_call(x_flat, cb_t)
    idx_flat = jnp.concatenate(
        [jnp.transpose(if1).reshape(-1), jnp.transpose(if2).reshape(-1)])
    table = codebook.reshape(N_Q * BINS, DIM)
    rows = _gather_call(table, idx_flat)
    rows1 = rows[:N_FRAMES * N_Q].reshape(N_Q, N_FRAMES, DIM)
    rows2 = rows[N_FRAMES * N_Q:].reshape(N_Q, N_FRAMES, DIM)
    indices, qst_flat, loss11 = _rescue_call(x_flat, rows1, rows2, i1, i2)
    quantized_st = jnp.transpose(qst_flat.reshape(b, t, c), (0, 2, 1))
    return quantized_st, indices, loss11[0, 0]
